# Initial kernel scaffold; baseline (speedup 1.0000x reference)
import jax
import jax.numpy as jnp
from jax.experimental import pallas as pl


def kernel(triangles, barycenters, indices_neigh_tri, number_neigh_tri):
    raise NotImplementedError("write your pallas kernel here")


# speedup vs baseline: 29.2926x; 29.2926x over previous
import functools

import jax
import jax.numpy as jnp
from jax import lax
from jax.experimental import pallas as pl
from jax.experimental.pallas import tpu as pltpu
from jax.experimental.pallas import tpu_sc as plsc

N = 100000
K = 17
F = 8
RB = 128
NP = 102400
NB = NP // RB
NW = 32
BPW = NB // NW
IDX = RB * K
CH = IDX // 128
IDXP = CH * 128
OW = (K - 1) * 5


def _feat_body(tri_ref, bary_ref, out_ref):
    t = tri_ref[...]
    d0 = t[:, 0:3] - t[:, 3:6]
    d1 = t[:, 0:3] - t[:, 6:9]
    d2 = t[:, 3:6] - t[:, 6:9]
    s0 = jnp.sum(d0 * d0, axis=1, keepdims=True)
    s1 = jnp.sum(d1 * d1, axis=1, keepdims=True)
    s2 = jnp.sum(d2 * d2, axis=1, keepdims=True)
    mn = jnp.sqrt(jnp.minimum(jnp.minimum(s0, s1), s2))
    mx = jnp.sqrt(jnp.maximum(jnp.maximum(s0, s1), s2))
    z = jnp.zeros_like(mn)
    out_ref[...] = jnp.concatenate([mn, mx, bary_ref[...], z, z, z], axis=1)


def _feat_table(tri9, bary):
    B = 1000
    return pl.pallas_call(
        _feat_body,
        grid=(N // B,),
        in_specs=[
            pl.BlockSpec((B, 9), lambda i: (i, 0)),
            pl.BlockSpec((B, 3), lambda i: (i, 0)),
        ],
        out_specs=pl.BlockSpec((B, F), lambda i: (i, 0)),
        out_shape=jax.ShapeDtypeStruct((N, F), jnp.float32),
    )(tri9, bary)


def _sc_body(feat_hbm, idxp_hbm, out_hbm, idx_v, g_v, out_v, sem):
    nc = 2
    wid = lax.axis_index("c") * 16 + lax.axis_index("s")

    def batch_body(m, carry):
        b = wid * BPW + m
        pltpu.sync_copy(idxp_hbm.at[b], idx_v)

        def fire(c, carry2):
            pltpu.async_copy(
                feat_hbm.at[idx_v.at[c]],
                g_v.at[pl.ds(c * 128, 128)],
                sem,
            )
            return carry2
        lax.fori_loop(0, CH, fire, 0)
        pltpu.make_async_copy(feat_hbm.at[pl.ds(0, IDXP)], g_v, sem).wait()

        def row_body(i, carry3):
            lane = lax.iota(jnp.int32, 16)
            base = i * K
            for t in range(5):
                p = t * 16 + lane
                jrow = base + 1 + p // 5
                jcol = p % 5
                cvals = plsc.load_gather(g_v, [jrow * 0 + base, jcol])
                nvals = plsc.load_gather(g_v, [jrow, jcol])
                out_v[i, pl.ds(t * 16, 16)] = cvals - nvals
            return carry3
        lax.fori_loop(0, RB, row_body, 0)

        pltpu.sync_copy(out_v, out_hbm.at[pl.ds(b * RB, RB)])
        return carry

    lax.fori_loop(0, BPW, batch_body, 0)


def _rmatrix_sc(feat, idxp):
    mesh = plsc.VectorSubcoreMesh(core_axis_name="c", subcore_axis_name="s")
    return pl.kernel(
        _sc_body,
        out_type=jax.ShapeDtypeStruct((NP, OW), jnp.float32),
        mesh=mesh,
        scratch_types=[
            pltpu.VMEM((CH, 128), jnp.int32),
            pltpu.VMEM((IDXP, F), jnp.float32),
            pltpu.VMEM((RB, OW), jnp.float32),
            pltpu.SemaphoreType.DMA,
        ],
        compiler_params=pltpu.CompilerParams(
            use_tc_tiling_on_sc=False, needs_layout_passes=False),
    )(feat, idxp)


def kernel(triangles, barycenters, indices_neigh_tri, number_neigh_tri):
    del number_neigh_tri
    tri9 = triangles.reshape(N, 9)
    feat = _feat_table(tri9, barycenters)
    idx32 = indices_neigh_tri.astype(jnp.int32)
    flat = idx32.reshape(N * K)
    idxp = jnp.pad(flat, (0, NP * K - N * K)).reshape(NB, CH, 128)
    out80 = _rmatrix_sc(feat, idxp)
    return out80[:N].reshape(N, K - 1, 5)


# speedup vs baseline: 32.4487x; 1.1077x over previous
import functools

import jax
import jax.numpy as jnp
from jax import lax
from jax.experimental import pallas as pl
from jax.experimental.pallas import tpu as pltpu
from jax.experimental.pallas import tpu_sc as plsc

N = 100000
K = 17
F = 8
RB = 125
NB = N // RB
NW = 32
BPW = NB // NW
IDX = RB * K
CH = 17
IDXP = CH * 128
OW = (K - 1) * 5


def _feat_body(tri_ref, bary_ref, out_ref):
    t = tri_ref[...]
    d0 = t[:, 0:3] - t[:, 3:6]
    d1 = t[:, 0:3] - t[:, 6:9]
    d2 = t[:, 3:6] - t[:, 6:9]
    s0 = jnp.sum(d0 * d0, axis=1, keepdims=True)
    s1 = jnp.sum(d1 * d1, axis=1, keepdims=True)
    s2 = jnp.sum(d2 * d2, axis=1, keepdims=True)
    mn = jnp.sqrt(jnp.minimum(jnp.minimum(s0, s1), s2))
    mx = jnp.sqrt(jnp.maximum(jnp.maximum(s0, s1), s2))
    z = jnp.zeros_like(mn)
    out_ref[...] = jnp.concatenate([mn, mx, bary_ref[...], z, z, z], axis=1)


def _feat_table(tri9, bary):
    B = 1000
    return pl.pallas_call(
        _feat_body,
        grid=(N // B,),
        in_specs=[
            pl.BlockSpec((B, 9), lambda i: (i, 0)),
            pl.BlockSpec((B, 3), lambda i: (i, 0)),
        ],
        out_specs=pl.BlockSpec((B, F), lambda i: (i, 0)),
        out_shape=jax.ShapeDtypeStruct((N, F), jnp.float32),
    )(tri9, bary)


def _sc_body(feat_hbm, idxp_hbm, out_hbm,
             idx0, idx1, g0, g1, o0, o1, gs0, gs1, os0, os1):
    wid = lax.axis_index("c") * 16 + lax.axis_index("s")
    b0 = wid * BPW

    def stage_and_fire(b, idx_v, g_v, gsem):
        pltpu.sync_copy(idxp_hbm.at[b], idx_v)

        def fire(c, carry2):
            pltpu.async_copy(
                feat_hbm.at[idx_v.at[c]],
                g_v.at[pl.ds(c * 128, 128)],
                gsem,
            )
            return carry2
        lax.fori_loop(0, CH, fire, 0)

    def compute(b, g_v, o_v, gsem, osem, first):
        pltpu.make_async_copy(feat_hbm.at[pl.ds(0, IDXP)], g_v, gsem).wait()

        @pl.when(jnp.logical_not(first))
        def _():
            pltpu.make_async_copy(
                out_hbm.at[pl.ds(0, RB)], o_v, osem).wait()

        def row_body(i, carry3):
            lane = lax.iota(jnp.int32, 16)
            bvec = jnp.full((16,), i * K, jnp.int32)
            for t in range(5):
                p = t * 16 + lane
                jrow = 1 + p // 5
                jcol = p % 5
                cvals = plsc.load_gather(g_v, [bvec, jcol])
                nvals = plsc.load_gather(g_v, [bvec + jrow, jcol])
                o_v[i, pl.ds(t * 16, 16)] = cvals - nvals
            return carry3
        lax.fori_loop(0, RB, row_body, 0)

        pltpu.async_copy(o_v, out_hbm.at[pl.ds(b * RB, RB)], osem)

    stage_and_fire(b0, idx0, g0, gs0)

    def pair_body(q, carry):
        tA = 2 * q

        @pl.when(tA + 1 < BPW)
        def _():
            stage_and_fire(b0 + tA + 1, idx1, g1, gs1)
        compute(b0 + tA, g0, o0, gs0, os0, q == 0)

        @pl.when(tA + 1 < BPW)
        def _():
            stage_and_fire(b0 + tA + 2, idx0, g0, gs0)
            compute(b0 + tA + 1, g1, o1, gs1, os1, q == 0)
        return carry

    lax.fori_loop(0, (BPW + 1) // 2, pair_body, 0)

    pltpu.make_async_copy(out_hbm.at[pl.ds(0, RB)], o0, os0).wait()

    @pl.when(BPW > 1)
    def _():
        pltpu.make_async_copy(out_hbm.at[pl.ds(0, RB)], o1, os1).wait()


def _rmatrix_sc(feat, idxp):
    mesh = plsc.VectorSubcoreMesh(core_axis_name="c", subcore_axis_name="s")
    return pl.kernel(
        _sc_body,
        out_type=jax.ShapeDtypeStruct((N, OW), jnp.float32),
        mesh=mesh,
        scratch_types=[
            pltpu.VMEM((CH, 128), jnp.int32),
            pltpu.VMEM((CH, 128), jnp.int32),
            pltpu.VMEM((IDXP, F), jnp.float32),
            pltpu.VMEM((IDXP, F), jnp.float32),
            pltpu.VMEM((RB, OW), jnp.float32),
            pltpu.VMEM((RB, OW), jnp.float32),
            pltpu.SemaphoreType.DMA,
            pltpu.SemaphoreType.DMA,
            pltpu.SemaphoreType.DMA,
            pltpu.SemaphoreType.DMA,
        ],
        compiler_params=pltpu.CompilerParams(
            use_tc_tiling_on_sc=False, needs_layout_passes=False),
    )(feat, idxp)


def kernel(triangles, barycenters, indices_neigh_tri, number_neigh_tri):
    del number_neigh_tri
    tri9 = triangles.reshape(N, 9)
    feat = _feat_table(tri9, barycenters)
    idx32 = indices_neigh_tri.astype(jnp.int32)
    flat = idx32.reshape(NB, IDX)
    idxp = jnp.pad(flat, ((0, 0), (0, IDXP - IDX))).reshape(NB, CH, 128)
    out80 = _rmatrix_sc(feat, idxp)
    return out80.reshape(N, K - 1, 5)
